# baseline (speedup 1.0000x reference)
import functools

import jax
import jax.numpy as jnp
from jax.experimental import pallas as pl
from jax.experimental.pallas import tpu as pltpu

_D = 768
_MAX_REL = 125
_VOCAB = 2 * _MAX_REL + 1
_CH = 2048
_NB = 2
_P = 4
_CP = _CH // _P


def _body(x_hbm, table_ref, out_hbm, xbuf, obuf, insems, outsems, *, S, N):
    nch = N // _CH
    tbl = table_ref[...]

    def load(i):
        sl = i % _NB
        return [pltpu.make_async_copy(
            x_hbm.at[pl.ds(i * _CH + p * _CP, _CP), :],
            xbuf.at[sl, pl.ds(p * _CP, _CP)], insems.at[sl, p])
            for p in range(_P)]

    def store(i):
        sl = i % _NB
        return [pltpu.make_async_copy(
            obuf.at[sl, pl.ds(p * _CP, _CP)],
            out_hbm.at[pl.ds(i * _CH + p * _CP, _CP), :], outsems.at[sl, p])
            for p in range(_P)]

    for i in range(min(_NB, nch)):
        for cp in load(i):
            cp.start()

    for i in range(nch):
        sl = i % _NB
        for cp in load(i):
            cp.wait()
        rows = i * _CH + jax.lax.broadcasted_iota(jnp.int32, (_CH, _VOCAB), 0)
        jj = jax.lax.bitwise_and(rows, S - 1)
        kk = jax.lax.broadcasted_iota(jnp.int32, (_CH, _VOCAB), 1)
        interior = ((kk >= 1) & (kk <= _VOCAB - 2)
                    & (jj >= kk - _MAX_REL) & (jj <= kk - _MAX_REL + S - 1))
        cnt = jnp.where(kk == 0, jnp.maximum(0, (S - _MAX_REL) - jj), 0)
        cnt = cnt + jnp.where(kk == _VOCAB - 1,
                              jnp.maximum(0, jj - (_MAX_REL - 1)), 0)
        cnt = cnt + interior.astype(jnp.int32)
        c = cnt.astype(jnp.float32) * (1.0 / S)
        pooled = jax.lax.dot_general(
            c, tbl,
            dimension_numbers=(((1,), (0,)), ((), ())),
            preferred_element_type=jnp.float32,
        )
        if i >= _NB:
            for cp in store(i - _NB):
                cp.wait()
        obuf[sl] = xbuf[sl] + pooled
        for cp in store(i):
            cp.start()
        if i + _NB < nch:
            for cp in load(i + _NB):
                cp.start()

    for i in range(max(0, nch - _NB), nch):
        for cp in store(i):
            cp.wait()


def kernel(x, table):
    B, S, d = x.shape
    V = table.shape[0]
    N = B * S
    xf = x.reshape(N, d)
    body = functools.partial(_body, S=S, N=N)
    out = pl.pallas_call(
        body,
        in_specs=[
            pl.BlockSpec(memory_space=pl.ANY),
            pl.BlockSpec((V, d), lambda: (0, 0)),
        ],
        out_specs=pl.BlockSpec(memory_space=pl.ANY),
        out_shape=jax.ShapeDtypeStruct((N, d), x.dtype),
        scratch_shapes=[
            pltpu.VMEM((_NB, _CH, d), jnp.float32),
            pltpu.VMEM((_NB, _CH, d), jnp.float32),
            pltpu.SemaphoreType.DMA((_NB, _P)),
            pltpu.SemaphoreType.DMA((_NB, _P)),
        ],
    )(xf, table)
    return out.reshape(B, S, d)


# speedup vs baseline: 1.0625x; 1.0625x over previous
import functools

import jax
import jax.numpy as jnp
from jax.experimental import pallas as pl
from jax.experimental.pallas import tpu as pltpu

_D = 768
_MAX_REL = 125
_VOCAB = 2 * _MAX_REL + 1
_CHS = 512
_NB = 3


def _body(x_hbm, table_ref, out_hbm, xbuf, obuf, insems, outsems, *, S, B):
    nch = S // _CHS
    tbl = table_ref[...]

    def load(i):
        sl = i % _NB
        return [pltpu.make_async_copy(
            x_hbm.at[b, pl.ds(i * _CHS, _CHS), :],
            xbuf.at[sl, b], insems.at[sl, b])
            for b in range(B)]

    def store(i):
        sl = i % _NB
        return [pltpu.make_async_copy(
            obuf.at[sl, b], out_hbm.at[b, pl.ds(i * _CHS, _CHS), :],
            outsems.at[sl, b])
            for b in range(B)]

    for i in range(min(_NB, nch)):
        for cp in load(i):
            cp.start()

    for i in range(nch):
        sl = i % _NB
        jj = i * _CHS + jax.lax.broadcasted_iota(jnp.int32, (_CHS, _VOCAB), 0)
        kk = jax.lax.broadcasted_iota(jnp.int32, (_CHS, _VOCAB), 1)
        interior = ((kk >= 1) & (kk <= _VOCAB - 2)
                    & (jj >= kk - _MAX_REL) & (jj <= kk - _MAX_REL + S - 1))
        cnt = jnp.where(kk == 0, jnp.maximum(0, (S - _MAX_REL) - jj), 0)
        cnt = cnt + jnp.where(kk == _VOCAB - 1,
                              jnp.maximum(0, jj - (_MAX_REL - 1)), 0)
        cnt = cnt + interior.astype(jnp.int32)
        c = cnt.astype(jnp.float32) * (1.0 / S)
        pooled = jax.lax.dot_general(
            c, tbl,
            dimension_numbers=(((1,), (0,)), ((), ())),
            preferred_element_type=jnp.float32,
        )
        for cp in load(i):
            cp.wait()
        if i >= _NB:
            for cp in store(i - _NB):
                cp.wait()
        obuf[sl] = xbuf[sl] + pooled[None, :, :]
        for cp in store(i):
            cp.start()
        if i + _NB < nch:
            for cp in load(i + _NB):
                cp.start()

    for i in range(max(0, nch - _NB), nch):
        for cp in store(i):
            cp.wait()


def kernel(x, table):
    B, S, d = x.shape
    V = table.shape[0]
    body = functools.partial(_body, S=S, B=B)
    return pl.pallas_call(
        body,
        in_specs=[
            pl.BlockSpec(memory_space=pl.ANY),
            pl.BlockSpec((V, d), lambda: (0, 0)),
        ],
        out_specs=pl.BlockSpec(memory_space=pl.ANY),
        out_shape=jax.ShapeDtypeStruct((B, S, d), x.dtype),
        scratch_shapes=[
            pltpu.VMEM((_NB, 2, _CHS, d), jnp.float32),
            pltpu.VMEM((_NB, 2, _CHS, d), jnp.float32),
            pltpu.SemaphoreType.DMA((_NB, 2)),
            pltpu.SemaphoreType.DMA((_NB, 2)),
        ],
    )(x, table)
